# baseline (speedup 1.0000x reference)
import jax
import jax.numpy as jnp
from jax.experimental import pallas as pl

_BM = 2048


def _router_block(x_ref, w_ref, logits_ref, mask_ref):
    logits = jax.lax.dot_general(
        x_ref[...],
        w_ref[...],
        dimension_numbers=(((1,), (1,)), ((), ())),
        preferred_element_type=jnp.float32,
    )
    logits_ref[...] = logits
    mask_ref[...] = logits > 0


def kernel(x, weight):
    m, k = x.shape
    e = weight.shape[0]
    logits, mask = pl.pallas_call(
        _router_block,
        grid=(m // _BM,),
        in_specs=[
            pl.BlockSpec((_BM, k), lambda i: (i, 0)),
            pl.BlockSpec((e, k), lambda i: (0, 0)),
        ],
        out_specs=[
            pl.BlockSpec((_BM, e), lambda i: (i, 0)),
            pl.BlockSpec((_BM, e), lambda i: (i, 0)),
        ],
        out_shape=[
            jax.ShapeDtypeStruct((m, e), jnp.float32),
            jax.ShapeDtypeStruct((m, e), jnp.bool_),
        ],
    )(x, weight)
    return (logits, mask)


# speedup vs baseline: 1.0109x; 1.0109x over previous
import jax
import jax.numpy as jnp
from jax.experimental import pallas as pl
from jax.experimental.pallas import tpu as pltpu

_BM = 1024


def _router_block(x_ref, w_ref, logits_ref, mask_ref):
    logits = jax.lax.dot_general(
        x_ref[...],
        w_ref[...],
        dimension_numbers=(((1,), (1,)), ((), ())),
        preferred_element_type=jnp.float32,
    )
    logits_ref[...] = logits
    mask_ref[...] = logits > 0


def kernel(x, weight):
    m, k = x.shape
    e = weight.shape[0]
    logits, mask = pl.pallas_call(
        _router_block,
        grid=(m // _BM,),
        in_specs=[
            pl.BlockSpec((_BM, k), lambda i: (i, 0)),
            pl.BlockSpec((e, k), lambda i: (0, 0)),
        ],
        out_specs=[
            pl.BlockSpec((_BM, e), lambda i: (i, 0)),
            pl.BlockSpec((_BM, e), lambda i: (i, 0)),
        ],
        out_shape=[
            jax.ShapeDtypeStruct((m, e), jnp.float32),
            jax.ShapeDtypeStruct((m, e), jnp.bool_),
        ],
        compiler_params=pltpu.CompilerParams(
            dimension_semantics=("parallel",),
        ),
    )(x, weight)
    return (logits, mask)
